# Initial kernel scaffold; baseline (speedup 1.0000x reference)
import jax
import jax.numpy as jnp
from jax.experimental import pallas as pl


def kernel(q_len, k_len, W):
    raise NotImplementedError("write your pallas kernel here")


# speedup vs baseline: 103.0091x; 103.0091x over previous
import jax
import jax.numpy as jnp
from jax.experimental import pallas as pl
from jax.experimental.pallas import tpu as pltpu

NUM_HEAD = 16
NUM_BUCKETS = 32
MAX_DISTANCE = 128
Q_LEN = 2048
K_LEN = 2048
WD = 4224
SW = 3968
BQ = 128


def _body(d_ref, wt_ref, out_ref, s_ref):
    m = pl.program_id(1)

    @pl.when(m == 0)
    def _init():
        d = d_ref[:, :]
        a = jnp.abs(d)
        large = 8.0 + jnp.round(jnp.log((a - 8).astype(jnp.float32)))
        mid = jnp.where(a < MAX_DISTANCE, large, 15.0)
        b = jnp.where(a <= 8, a.astype(jnp.float32), mid)
        b = b + jnp.where(d > 0, 16.0, 0.0)
        bi = b.astype(jnp.int32)
        wrow = wt_ref[0]
        acc = jnp.zeros((1, WD), jnp.float32)
        for bb in range(NUM_BUCKETS):
            acc = acc + jnp.where(bi == bb, wrow[:, bb:bb + 1], 0.0)
        for k in range(16):
            rows = [
                jax.lax.slice(acc, (0, 127 - 8 * k - s3), (1, 127 - 8 * k - s3 + SW))
                for s3 in range(8)
            ]
            s_ref[8 * k:8 * k + 8, :] = jnp.concatenate(rows, axis=0)

    c0 = pl.multiple_of((Q_LEN - BQ) - BQ * m, BQ)
    out_ref[0, 0, :, :] = s_ref[:, pl.ds(c0, K_LEN)]


def kernel(q_len, k_len, W):
    delta = jnp.asarray(q_len - k_len, jnp.int32)
    t = jax.lax.broadcasted_iota(jnp.int32, (1, WD), 1)
    d_row = (Q_LEN - 1) - t + delta
    wt = W.T.astype(jnp.float32).reshape(NUM_HEAD, 1, NUM_BUCKETS)

    out = pl.pallas_call(
        _body,
        grid=(NUM_HEAD, Q_LEN // BQ),
        in_specs=[
            pl.BlockSpec((1, WD), lambda h, m: (0, 0)),
            pl.BlockSpec((1, 1, NUM_BUCKETS), lambda h, m: (h, 0, 0)),
        ],
        out_specs=pl.BlockSpec((1, 1, BQ, K_LEN), lambda h, m: (0, h, m, 0)),
        out_shape=jax.ShapeDtypeStruct((1, NUM_HEAD, Q_LEN, K_LEN), jnp.float32),
        scratch_shapes=[pltpu.VMEM((BQ, SW), jnp.float32)],
    )(d_row, wt)
    return out


# speedup vs baseline: 168.0208x; 1.6311x over previous
import jax
import jax.numpy as jnp
from jax.experimental import pallas as pl
from jax.experimental.pallas import tpu as pltpu

NUM_HEAD = 16
NUM_BUCKETS = 32
MAX_DISTANCE = 128
Q_LEN = 2048
K_LEN = 2048
WD = 4224
SW = 3968
BQ = 512


def _body(d_ref, wt_ref, out_ref, s_ref):
    m = pl.program_id(1)

    @pl.when(m == 0)
    def _init():
        d = d_ref[:, :]
        a = jnp.abs(d)
        large = 8.0 + jnp.round(jnp.log((a - 8).astype(jnp.float32)))
        mid = jnp.where(a < MAX_DISTANCE, large, 15.0)
        b = jnp.where(a <= 8, a.astype(jnp.float32), mid)
        b = b + jnp.where(d > 0, 16.0, 0.0)
        bi = b.astype(jnp.int32)
        wrow = wt_ref[0]
        acc = jnp.zeros((1, WD), jnp.float32)
        for bb in range(NUM_BUCKETS):
            acc = acc + jnp.where(bi == bb, wrow[:, bb:bb + 1], 0.0)
        for k in range(16):
            rows = [
                jax.lax.slice(acc, (0, 127 - 8 * k - s3), (1, 127 - 8 * k - s3 + SW))
                for s3 in range(8)
            ]
            s_ref[8 * k:8 * k + 8, :] = jnp.concatenate(rows, axis=0)

    for g in range(BQ // 128):
        c0 = pl.multiple_of((Q_LEN - 128) - BQ * m - 128 * g, 128)
        out_ref[0, 0, 128 * g:128 * g + 128, :] = s_ref[:, pl.ds(c0, K_LEN)]


def kernel(q_len, k_len, W):
    delta = jnp.asarray(q_len - k_len, jnp.int32)
    t = jax.lax.broadcasted_iota(jnp.int32, (1, WD), 1)
    d_row = (Q_LEN - 1) - t + delta
    wt = W.T.astype(jnp.float32).reshape(NUM_HEAD, 1, NUM_BUCKETS)

    out = pl.pallas_call(
        _body,
        grid=(NUM_HEAD, Q_LEN // BQ),
        in_specs=[
            pl.BlockSpec((1, WD), lambda h, m: (0, 0)),
            pl.BlockSpec((1, 1, NUM_BUCKETS), lambda h, m: (h, 0, 0)),
        ],
        out_specs=pl.BlockSpec((1, 1, BQ, K_LEN), lambda h, m: (0, h, m, 0)),
        out_shape=jax.ShapeDtypeStruct((1, NUM_HEAD, Q_LEN, K_LEN), jnp.float32),
        scratch_shapes=[pltpu.VMEM((128, SW), jnp.float32)],
    )(d_row, wt)
    return out


# speedup vs baseline: 185.5494x; 1.1043x over previous
import jax
import jax.numpy as jnp
from jax.experimental import pallas as pl
from jax.experimental.pallas import tpu as pltpu

NUM_HEAD = 16
NUM_BUCKETS = 32
MAX_DISTANCE = 128
Q_LEN = 2048
K_LEN = 2048
WD = 4224
SW = 3968
BQ = 1024


def _body(d_ref, wt_ref, out_ref, s_ref):
    m = pl.program_id(1)

    @pl.when(m == 0)
    def _init():
        d = d_ref[:, :]
        a = jnp.abs(d)
        large = 8.0 + jnp.round(jnp.log((a - 8).astype(jnp.float32)))
        mid = jnp.where(a < MAX_DISTANCE, large, 15.0)
        b = jnp.where(a <= 8, a.astype(jnp.float32), mid)
        b = b + jnp.where(d > 0, 16.0, 0.0)
        bi = b.astype(jnp.int32)
        wrow = wt_ref[0]
        acc = jnp.zeros((1, WD), jnp.float32)
        for bb in range(NUM_BUCKETS):
            acc = acc + jnp.where(bi == bb, wrow[:, bb:bb + 1], 0.0)
        for k in range(16):
            rows = [
                jax.lax.slice(acc, (0, 127 - 8 * k - s3), (1, 127 - 8 * k - s3 + SW))
                for s3 in range(8)
            ]
            s_ref[8 * k:8 * k + 8, :] = jnp.concatenate(rows, axis=0)

    for g in range(BQ // 128):
        c0 = pl.multiple_of((Q_LEN - 128) - BQ * m - 128 * g, 128)
        out_ref[0, 0, 128 * g:128 * g + 128, :] = s_ref[:, pl.ds(c0, K_LEN)]


def kernel(q_len, k_len, W):
    delta = jnp.asarray(q_len - k_len, jnp.int32)
    t = jax.lax.broadcasted_iota(jnp.int32, (1, WD), 1)
    d_row = (Q_LEN - 1) - t + delta
    wt = W.T.astype(jnp.float32).reshape(NUM_HEAD, 1, NUM_BUCKETS)

    out = pl.pallas_call(
        _body,
        grid=(NUM_HEAD, Q_LEN // BQ),
        in_specs=[
            pl.BlockSpec((1, WD), lambda h, m: (0, 0)),
            pl.BlockSpec((1, 1, NUM_BUCKETS), lambda h, m: (h, 0, 0)),
        ],
        out_specs=pl.BlockSpec((1, 1, BQ, K_LEN), lambda h, m: (0, h, m, 0)),
        out_shape=jax.ShapeDtypeStruct((1, NUM_HEAD, Q_LEN, K_LEN), jnp.float32),
        scratch_shapes=[pltpu.VMEM((128, SW), jnp.float32)],
    )(d_row, wt)
    return out


# speedup vs baseline: 188.0831x; 1.0137x over previous
import jax
import jax.numpy as jnp
from jax.experimental import pallas as pl
from jax.experimental.pallas import tpu as pltpu

NUM_HEAD = 16
NUM_BUCKETS = 32
MAX_DISTANCE = 128
Q_LEN = 2048
K_LEN = 2048
WD = 4224
SW = 3968
BQ = 2048


def _body(d_ref, wt_ref, out_ref, s_ref):
    m = pl.program_id(1)

    @pl.when(m == 0)
    def _init():
        d = d_ref[:, :]
        a = jnp.abs(d)
        large = 8.0 + jnp.round(jnp.log((a - 8).astype(jnp.float32)))
        mid = jnp.where(a < MAX_DISTANCE, large, 15.0)
        b = jnp.where(a <= 8, a.astype(jnp.float32), mid)
        b = b + jnp.where(d > 0, 16.0, 0.0)
        bi = b.astype(jnp.int32)
        wrow = wt_ref[0]
        acc = jnp.zeros((1, WD), jnp.float32)
        for bb in range(NUM_BUCKETS):
            acc = acc + jnp.where(bi == bb, wrow[:, bb:bb + 1], 0.0)
        for k in range(16):
            rows = [
                jax.lax.slice(acc, (0, 127 - 8 * k - s3), (1, 127 - 8 * k - s3 + SW))
                for s3 in range(8)
            ]
            s_ref[8 * k:8 * k + 8, :] = jnp.concatenate(rows, axis=0)

    for g in range(BQ // 128):
        c0 = pl.multiple_of((Q_LEN - 128) - BQ * m - 128 * g, 128)
        out_ref[0, 0, 128 * g:128 * g + 128, :] = s_ref[:, pl.ds(c0, K_LEN)]


def kernel(q_len, k_len, W):
    delta = jnp.asarray(q_len - k_len, jnp.int32)
    t = jax.lax.broadcasted_iota(jnp.int32, (1, WD), 1)
    d_row = (Q_LEN - 1) - t + delta
    wt = W.T.astype(jnp.float32).reshape(NUM_HEAD, 1, NUM_BUCKETS)

    out = pl.pallas_call(
        _body,
        grid=(NUM_HEAD, Q_LEN // BQ),
        in_specs=[
            pl.BlockSpec((1, WD), lambda h, m: (0, 0)),
            pl.BlockSpec((1, 1, NUM_BUCKETS), lambda h, m: (h, 0, 0)),
        ],
        out_specs=pl.BlockSpec((1, 1, BQ, K_LEN), lambda h, m: (0, h, m, 0)),
        out_shape=jax.ShapeDtypeStruct((1, NUM_HEAD, Q_LEN, K_LEN), jnp.float32),
        scratch_shapes=[pltpu.VMEM((128, SW), jnp.float32)],
    )(d_row, wt)
    return out


# speedup vs baseline: 188.9492x; 1.0046x over previous
import jax
import jax.numpy as jnp
from jax.experimental import pallas as pl
from jax.experimental.pallas import tpu as pltpu

NUM_HEAD = 16
NUM_BUCKETS = 32
MAX_DISTANCE = 128
Q_LEN = 2048
K_LEN = 2048
WD = 4224
SW = 3968
BQ = 2048


def _body(d_ref, wt_ref, out_ref, s_ref):
    m = pl.program_id(1)

    @pl.when(m == 0)
    def _init():
        d = d_ref[:, :]
        a = jnp.abs(d)
        large = 8.0 + jnp.round(jnp.log((a - 8).astype(jnp.float32)))
        mid = jnp.where(a < MAX_DISTANCE, large, 15.0)
        b = jnp.where(a <= 8, a.astype(jnp.float32), mid)
        b = b + jnp.where(d > 0, 16.0, 0.0)
        bi = b.astype(jnp.int32)
        wrow = wt_ref[0]
        acc = jnp.zeros((1, WD), jnp.float32)
        for bb in range(NUM_BUCKETS):
            acc = acc + jnp.where(bi == bb, wrow[:, bb:bb + 1], 0.0)
        for k in range(16):
            rows = [
                jax.lax.slice(acc, (0, 127 - 8 * k - s3), (1, 127 - 8 * k - s3 + SW))
                for s3 in range(8)
            ]
            s_ref[8 * k:8 * k + 8, :] = jnp.concatenate(rows, axis=0)

    for g in range(BQ // 128):
        c0 = pl.multiple_of((Q_LEN - 128) - BQ * m - 128 * g, 128)
        out_ref[0, 0, 128 * g:128 * g + 128, :] = s_ref[:, pl.ds(c0, K_LEN)]


def kernel(q_len, k_len, W):
    delta = jnp.asarray(q_len - k_len, jnp.int32)
    t = jax.lax.broadcasted_iota(jnp.int32, (1, WD), 1)
    d_row = (Q_LEN - 1) - t + delta
    wt = W.T.astype(jnp.float32).reshape(NUM_HEAD, 1, NUM_BUCKETS)

    out = pl.pallas_call(
        _body,
        grid=(NUM_HEAD, Q_LEN // BQ),
        in_specs=[
            pl.BlockSpec((1, WD), lambda h, m: (0, 0)),
            pl.BlockSpec((1, 1, NUM_BUCKETS), lambda h, m: (h, 0, 0)),
        ],
        out_specs=pl.BlockSpec((1, 1, BQ, K_LEN), lambda h, m: (0, h, m, 0)),
        out_shape=jax.ShapeDtypeStruct((1, NUM_HEAD, Q_LEN, K_LEN), jnp.float32),
        scratch_shapes=[pltpu.VMEM((128, SW), jnp.float32)],
        compiler_params=pltpu.CompilerParams(
            dimension_semantics=("parallel", "arbitrary")),
    )(d_row, wt)
    return out
